# baseline (speedup 1.0000x reference)
import functools

import jax
import jax.numpy as jnp
from jax import lax
from jax.experimental import pallas as pl
from jax.experimental.pallas import tpu as pltpu
from jax.experimental.pallas import tpu_sc as plsc

B = 1024
D = 16
N_ITEMS = 100000
NC = 2
NS = 16
NW = NC * NS
B_PER_W = B // NW

IB = 2048


@functools.partial(
    pl.kernel,
    out_type=jax.ShapeDtypeStruct((B, D), jnp.float32),
    mesh=plsc.VectorSubcoreMesh(core_axis_name="c", subcore_axis_name="s"),
    compiler_params=pltpu.CompilerParams(use_tc_tiling_on_sc=False),
    scratch_types=[
        pltpu.VMEM((B_PER_W,), jnp.int32),
        pltpu.VMEM((B_PER_W, D), jnp.float32),
        pltpu.SemaphoreType.DMA,
    ],
)
def _sc_gather(table_hbm, idx_hbm, out_hbm, idx_v, rows_v, sem):
    wid = lax.axis_index("s") * NC + lax.axis_index("c")
    base = wid * B_PER_W
    pltpu.sync_copy(idx_hbm.at[pl.ds(base, B_PER_W)], idx_v)
    pltpu.async_copy(table_hbm.at[idx_v], rows_v, sem).wait()
    pltpu.sync_copy(rows_v, out_hbm.at[pl.ds(base, B_PER_W)])


def _mm_body(u_ref, it_ref, out_ref):
    out_ref[...] = lax.dot_general(
        u_ref[...], it_ref[...],
        dimension_numbers=(((1,), (1,)), ((), ())),
        preferred_element_type=jnp.float32,
    )


def _tc_matmul(u, item_table):
    return pl.pallas_call(
        _mm_body,
        grid=(pl.cdiv(N_ITEMS, IB),),
        in_specs=[
            pl.BlockSpec((B, D), lambda i: (0, 0)),
            pl.BlockSpec((IB, D), lambda i: (i, 0)),
        ],
        out_specs=pl.BlockSpec((B, IB), lambda i: (0, i)),
        out_shape=jax.ShapeDtypeStruct((B, N_ITEMS), jnp.float32),
    )(u, item_table)


@jax.jit
def kernel(users, user_table, item_table):
    idx = users.reshape(-1).astype(jnp.int32)
    u = _sc_gather(user_table, idx)
    return _tc_matmul(u, item_table)


# speedup vs baseline: 1.0047x; 1.0047x over previous
import functools

import jax
import jax.numpy as jnp
from jax import lax
from jax.experimental import pallas as pl
from jax.experimental.pallas import tpu as pltpu
from jax.experimental.pallas import tpu_sc as plsc

B = 1024
D = 16
N_ITEMS = 100000
NC = 2
NS = 16
NW = NC * NS
B_PER_W = B // NW
ROWS_PER_LINE = 128 // D
N_USERS_LINES = 1000000 // ROWS_PER_LINE

IB = 2048


@functools.partial(
    pl.kernel,
    out_type=jax.ShapeDtypeStruct((B, D), jnp.float32),
    mesh=plsc.VectorSubcoreMesh(core_axis_name="c", subcore_axis_name="s"),
    compiler_params=pltpu.CompilerParams(needs_layout_passes=False),
    scratch_types=[
        pltpu.VMEM((B_PER_W,), jnp.int32),
        pltpu.VMEM((B_PER_W,), jnp.int32),
        pltpu.VMEM((B_PER_W, 128), jnp.float32),
        pltpu.VMEM((B_PER_W, D), jnp.float32),
        pltpu.SemaphoreType.DMA,
    ],
)
def _sc_gather(table128, q_hbm, off_hbm, out_hbm,
               q_v, off_v, lines_v, out_v, sem):
    wid = lax.axis_index("s") * NC + lax.axis_index("c")
    base = wid * B_PER_W
    pltpu.sync_copy(q_hbm.at[pl.ds(base, B_PER_W)], q_v)
    pltpu.sync_copy(off_hbm.at[pl.ds(base, B_PER_W)], off_v)
    pltpu.async_copy(table128.at[q_v], lines_v, sem).wait()
    lane = lax.iota(jnp.int32, 16)
    for g in range(B_PER_W // 16):
        row = lane + g * 16
        off16 = off_v[pl.ds(g * 16, 16)]
        for dd in range(D):
            vals = plsc.load_gather(lines_v, [row, off16 + dd])
            plsc.store_scatter(out_v, [row, jnp.full((16,), dd, jnp.int32)], vals)
    pltpu.sync_copy(out_v, out_hbm.at[pl.ds(base, B_PER_W)])


def _mm_body(u_ref, it_ref, out_ref):
    out_ref[...] = lax.dot_general(
        u_ref[...], it_ref[...],
        dimension_numbers=(((1,), (1,)), ((), ())),
        preferred_element_type=jnp.float32,
    )


def _tc_matmul(u, item_table):
    return pl.pallas_call(
        _mm_body,
        grid=(pl.cdiv(N_ITEMS, IB),),
        in_specs=[
            pl.BlockSpec((B, D), lambda i: (0, 0)),
            pl.BlockSpec((IB, D), lambda i: (i, 0)),
        ],
        out_specs=pl.BlockSpec((B, IB), lambda i: (0, i)),
        out_shape=jax.ShapeDtypeStruct((B, N_ITEMS), jnp.float32),
    )(u, item_table)


@jax.jit
def kernel(users, user_table, item_table):
    idx = users.reshape(-1).astype(jnp.int32)
    q = idx // ROWS_PER_LINE
    off = (idx % ROWS_PER_LINE) * D
    table128 = user_table.reshape(N_USERS_LINES, 128)
    u = _sc_gather(table128, q, off)
    return _tc_matmul(u, item_table)
